# Initial kernel scaffold; baseline (speedup 1.0000x reference)
import jax
import jax.numpy as jnp
from jax.experimental import pallas as pl


def kernel(seq_len, rel_emb):
    raise NotImplementedError("write your pallas kernel here")


# speedup vs baseline: 41.9776x; 41.9776x over previous
import functools

import jax
import jax.numpy as jnp
from jax import lax
from jax.experimental import pallas as pl
from jax.experimental.pallas import tpu as pltpu
from jax.experimental.pallas import tpu_sc as plsc

_MAXD = 128
_H = 16
_S = 2048
_T = 2 * _MAXD + 1
_EW = 2 * _S - 1
_WPAD = 4096
_NTILES = 32
_ROWS_PER_TILE = _H * _S // _NTILES
_GROUP = 8


def _sc_expand(e8):
    mesh = plsc.VectorSubcoreMesh(core_axis_name="c", subcore_axis_name="s")

    @functools.partial(
        pl.kernel,
        out_type=jax.ShapeDtypeStruct((_H * _S * _S,), jnp.float32),
        mesh=mesh,
        scratch_types=[
            pltpu.VMEM((8 * _WPAD,), jnp.float32),
            pltpu.SemaphoreType.DMA,
        ],
    )
    def expand(e8_hbm, out_hbm, e8_v, sem):
        wid = lax.axis_index("s") * 2 + lax.axis_index("c")
        head = wid // 2
        half = wid % 2
        pltpu.sync_copy(e8_hbm.at[head], e8_v)

        def group_body(g, carry):
            copies = []
            for b in range(_GROUP):
                i_local = half * _ROWS_PER_TILE + g * _GROUP + b
                start = (_S - 1) - i_local
                r = lax.rem(start, 8)
                off = start - r
                src_off = pl.multiple_of(r * _WPAD + off, 8)
                src = e8_v.at[pl.ds(src_off, _S)]
                dst_off = pl.multiple_of((head * _S + i_local) * _S, 8)
                dst = out_hbm.at[pl.ds(dst_off, _S)]
                copies.append(pltpu.async_copy(src, dst, sem))
            for c in copies:
                c.wait()
            return carry

        lax.fori_loop(0, _ROWS_PER_TILE // _GROUP, group_body, 0)

    return expand(e8)


def kernel(seq_len, rel_emb):
    del seq_len
    e = jnp.concatenate(
        [
            jnp.broadcast_to(rel_emb[0], (_S - 1 - _MAXD, _H)),
            rel_emb,
            jnp.broadcast_to(rel_emb[_T - 1], (_S - 1 - _MAXD, _H)),
        ],
        axis=0,
    )
    e_pad = jnp.pad(e, ((0, _WPAD + 8 - _EW), (0, 0)))
    e8 = jnp.stack([e_pad[r:r + _WPAD] for r in range(8)], 0)
    e8 = jnp.transpose(e8, (2, 0, 1)).reshape(_H, 8 * _WPAD)
    out = _sc_expand(e8)
    return out.reshape(_H, _S, _S)


# speedup vs baseline: 42.2666x; 1.0069x over previous
import functools

import jax
import jax.numpy as jnp
from jax import lax
from jax.experimental import pallas as pl
from jax.experimental.pallas import tpu as pltpu
from jax.experimental.pallas import tpu_sc as plsc

_MAXD = 128
_H = 16
_S = 2048
_T = 2 * _MAXD + 1
_EW = 2 * _S - 1
_WPAD = 4096
_NTILES = 32
_ROWS_PER_TILE = _H * _S // _NTILES
_GROUP = 8


def _sc_expand(e8):
    mesh = plsc.VectorSubcoreMesh(core_axis_name="c", subcore_axis_name="s")

    @functools.partial(
        pl.kernel,
        out_type=jax.ShapeDtypeStruct((_H * _S * _S,), jnp.float32),
        mesh=mesh,
        scratch_types=[
            pltpu.VMEM((8 * _WPAD,), jnp.float32),
            pltpu.SemaphoreType.DMA,
        ],
    )
    def expand(e8_hbm, out_hbm, e8_v, sem):
        wid = lax.axis_index("s") * 2 + lax.axis_index("c")
        head = wid // 2
        half = wid % 2
        pltpu.sync_copy(e8_hbm.at[head], e8_v)

        def fire_group(g):
            copies = []
            for b in range(_GROUP):
                i_local = half * _ROWS_PER_TILE + g * _GROUP + b
                start = (_S - 1) - i_local
                r = lax.rem(start, 8)
                off = start - r
                src_off = pl.multiple_of(r * _WPAD + off, 8)
                src = e8_v.at[pl.ds(src_off, _S)]
                dst_off = pl.multiple_of((head * _S + i_local) * _S, 8)
                dst = out_hbm.at[pl.ds(dst_off, _S)]
                copies.append(pltpu.async_copy(src, dst, sem))
            return copies

        n_groups = _ROWS_PER_TILE // _GROUP

        def group_body(g, carry):
            for c in fire_group(g + 1):
                c.wait()
            return carry

        fire_group(0)
        lax.fori_loop(0, n_groups - 1, group_body, 0)
        tail = fire_group(0)
        for c in tail:
            c.wait()
        for c in tail:
            c.wait()

    return expand(e8)


def kernel(seq_len, rel_emb):
    del seq_len
    e = jnp.concatenate(
        [
            jnp.broadcast_to(rel_emb[0], (_S - 1 - _MAXD, _H)),
            rel_emb,
            jnp.broadcast_to(rel_emb[_T - 1], (_S - 1 - _MAXD, _H)),
        ],
        axis=0,
    )
    e_pad = jnp.pad(e, ((0, _WPAD + 8 - _EW), (0, 0)))
    e8 = jnp.stack([e_pad[r:r + _WPAD] for r in range(8)], 0)
    e8 = jnp.transpose(e8, (2, 0, 1)).reshape(_H, 8 * _WPAD)
    out = _sc_expand(e8)
    return out.reshape(_H, _S, _S)
